# Initial kernel scaffold; baseline (speedup 1.0000x reference)
import jax
import jax.numpy as jnp
from jax.experimental import pallas as pl


def kernel(log_v0, log_vt, t, batch, log_alphas_v, log_one_minus_alphas_v, log_alphas_cumprod_v, log_one_minus_alphas_cumprod_v, prior_probs):
    raise NotImplementedError("write your pallas kernel here")


# speedup vs baseline: 4.6978x; 4.6978x over previous
import functools

import jax
import jax.numpy as jnp
from jax import lax
from jax.experimental import pallas as pl
from jax.experimental.pallas import tpu as pltpu
from jax.experimental.pallas import tpu_sc as plsc

NC = 2
NS = 16
LANES = 16
NW = NC * NS


_SC_CHUNK = 512


def _sc_coef_body(n_rows, t_len, t_hbm, batch_hbm, la_hbm, l1ma_hbm, lac_hbm,
                  l1mac_hbm, coef_hbm, t_v, batch_v, la_v, l1ma_v, lac_v,
                  l1mac_v, coef_v):
    rows = n_rows // NW
    chunk = min(rows, _SC_CHUNK)
    wid = lax.axis_index("s") * NC + lax.axis_index("c")
    base = wid * rows
    pltpu.sync_copy(t_hbm, t_v)
    pltpu.sync_copy(la_hbm, la_v)
    pltpu.sync_copy(l1ma_hbm, l1ma_v)
    pltpu.sync_copy(lac_hbm, lac_v)
    pltpu.sync_copy(l1mac_hbm, l1mac_v)

    zero = jnp.zeros((LANES,), jnp.int32)
    for ck in range(rows // chunk):
        cbase = base + ck * chunk
        pltpu.sync_copy(batch_hbm.at[pl.ds(cbase, chunk)], batch_v)
        for j in range(chunk // LANES):
            bv = batch_v[pl.ds(j * LANES, LANES)]
            tv = plsc.load_gather(t_v, [bv])
            tm1 = jnp.maximum(tv - 1, 0)
            a = plsc.load_gather(lac_v, [tm1])
            c = plsc.load_gather(l1mac_v, [tm1])
            d = plsc.load_gather(la_v, [tv])
            e = plsc.load_gather(l1ma_v, [tv])
            ridx = j * LANES + lax.iota(jnp.int32, LANES)
            plsc.store_scatter(coef_v, [ridx, zero], a)
            plsc.store_scatter(coef_v, [ridx, zero + 1], c)
            plsc.store_scatter(coef_v, [ridx, zero + 2], d)
            plsc.store_scatter(coef_v, [ridx, zero + 3], e)
        pltpu.sync_copy(coef_v, coef_hbm.at[pl.ds(cbase, chunk)])


def _sc_coef(t, batch, la, l1ma, lac, l1mac):
    n_rows = batch.shape[0]
    b = t.shape[0]
    t_len = la.shape[0]
    rows = n_rows // NW
    mesh = plsc.VectorSubcoreMesh(core_axis_name="c", subcore_axis_name="s",
                                  num_cores=NC, num_subcores=NS)
    body = functools.partial(_sc_coef_body, n_rows, t_len)
    return pl.kernel(
        body,
        out_type=jax.ShapeDtypeStruct((n_rows, 4), jnp.float32),
        mesh=mesh,
        scratch_types=[
            pltpu.VMEM((b,), jnp.int32),
            pltpu.VMEM((min(rows, _SC_CHUNK),), jnp.int32),
            pltpu.VMEM((t_len,), jnp.float32),
            pltpu.VMEM((t_len,), jnp.float32),
            pltpu.VMEM((t_len,), jnp.float32),
            pltpu.VMEM((t_len,), jnp.float32),
            pltpu.VMEM((min(rows, _SC_CHUNK), 4), jnp.float32),
        ],
        compiler_params=pltpu.CompilerParams(needs_layout_passes=False),
    )(t, batch, la, l1ma, lac, l1mac)


def _tc_body(v0_ref, vt_ref, coef_ref, prior_ref, out_ref):
    p = prior_ref[...]
    a = coef_ref[:, 0:1]
    c = coef_ref[:, 1:2]
    d = coef_ref[:, 2:3]
    e = coef_ref[:, 3:4]
    x1 = v0_ref[...] + a
    y1 = c + p
    la1 = jnp.maximum(x1, y1) + jnp.log1p(jnp.exp(-jnp.abs(x1 - y1)))
    x2 = vt_ref[...] + d
    y2 = e + p
    la2 = jnp.maximum(x2, y2) + jnp.log1p(jnp.exp(-jnp.abs(x2 - y2)))
    u = la1 + la2
    m = jnp.max(u, axis=1, keepdims=True)
    lse = m + jnp.log(jnp.sum(jnp.exp(u - m), axis=1, keepdims=True))
    out_ref[...] = u - lse


def _tc_dense(log_v0, log_vt, coef, prior, block=256):
    n, k = log_v0.shape
    grid = (n // block,)
    return pl.pallas_call(
        _tc_body,
        grid=grid,
        in_specs=[
            pl.BlockSpec((block, k), lambda i: (i, 0)),
            pl.BlockSpec((block, k), lambda i: (i, 0)),
            pl.BlockSpec((block, 4), lambda i: (i, 0)),
            pl.BlockSpec((1, k), lambda i: (0, 0)),
        ],
        out_specs=pl.BlockSpec((block, k), lambda i: (i, 0)),
        out_shape=jax.ShapeDtypeStruct((n, k), jnp.float32),
        compiler_params=pltpu.CompilerParams(
            dimension_semantics=("arbitrary",),
        ),
    )(log_v0, log_vt, coef, prior)


def kernel(log_v0, log_vt, t, batch, log_alphas_v, log_one_minus_alphas_v,
           log_alphas_cumprod_v, log_one_minus_alphas_cumprod_v, prior_probs):
    t_len = log_alphas_v.shape[0]
    pad_to = ((t_len + LANES - 1) // LANES) * LANES

    def padt(v):
        return jnp.concatenate(
            [v, jnp.zeros((pad_to - t_len,), v.dtype)]) if pad_to > t_len else v

    coef = _sc_coef(t, batch, padt(log_alphas_v),
                    padt(log_one_minus_alphas_v),
                    padt(log_alphas_cumprod_v),
                    padt(log_one_minus_alphas_cumprod_v))
    return _tc_dense(log_v0, log_vt, coef, prior_probs)


# speedup vs baseline: 5.6307x; 1.1986x over previous
import functools

import jax
import jax.numpy as jnp
from jax import lax
from jax.experimental import pallas as pl
from jax.experimental.pallas import tpu as pltpu
from jax.experimental.pallas import tpu_sc as plsc

NC = 2
NS = 16
LANES = 16
NW = NC * NS


_SC_CHUNK = 512


def _sc_coef_body(n_rows, t_len, t_hbm, batch_hbm, la_hbm, l1ma_hbm, lac_hbm,
                  l1mac_hbm, coef_hbm, t_v, batch_v, la_v, l1ma_v, lac_v,
                  l1mac_v, coef_v):
    rows = n_rows // NW
    chunk = min(rows, _SC_CHUNK)
    wid = lax.axis_index("s") * NC + lax.axis_index("c")
    base = wid * rows
    pltpu.sync_copy(t_hbm, t_v)
    pltpu.sync_copy(la_hbm, la_v)
    pltpu.sync_copy(l1ma_hbm, l1ma_v)
    pltpu.sync_copy(lac_hbm, lac_v)
    pltpu.sync_copy(l1mac_hbm, l1mac_v)

    zero = jnp.zeros((LANES,), jnp.int32)
    for ck in range(rows // chunk):
        cbase = base + ck * chunk
        pltpu.sync_copy(batch_hbm.at[pl.ds(cbase, chunk)], batch_v)
        for j in range(chunk // LANES):
            bv = batch_v[pl.ds(j * LANES, LANES)]
            tv = plsc.load_gather(t_v, [bv])
            tm1 = jnp.maximum(tv - 1, 0)
            a = plsc.load_gather(lac_v, [tm1])
            c = plsc.load_gather(l1mac_v, [tm1])
            d = plsc.load_gather(la_v, [tv])
            e = plsc.load_gather(l1ma_v, [tv])
            ridx = j * LANES + lax.iota(jnp.int32, LANES)
            plsc.store_scatter(coef_v, [ridx, zero], a - c)
            plsc.store_scatter(coef_v, [ridx, zero + 1], d - e)
        pltpu.sync_copy(coef_v, coef_hbm.at[pl.ds(cbase, chunk)])


def _sc_coef(t, batch, la, l1ma, lac, l1mac):
    n_rows = batch.shape[0]
    b = t.shape[0]
    t_len = la.shape[0]
    rows = n_rows // NW
    mesh = plsc.VectorSubcoreMesh(core_axis_name="c", subcore_axis_name="s",
                                  num_cores=NC, num_subcores=NS)
    body = functools.partial(_sc_coef_body, n_rows, t_len)
    return pl.kernel(
        body,
        out_type=jax.ShapeDtypeStruct((n_rows, 2), jnp.float32),
        mesh=mesh,
        scratch_types=[
            pltpu.VMEM((b,), jnp.int32),
            pltpu.VMEM((min(rows, _SC_CHUNK),), jnp.int32),
            pltpu.VMEM((t_len,), jnp.float32),
            pltpu.VMEM((t_len,), jnp.float32),
            pltpu.VMEM((t_len,), jnp.float32),
            pltpu.VMEM((t_len,), jnp.float32),
            pltpu.VMEM((min(rows, _SC_CHUNK), 2), jnp.float32),
        ],
        compiler_params=pltpu.CompilerParams(needs_layout_passes=False),
    )(t, batch, la, l1ma, lac, l1mac)


def _tc_body(v0_ref, vt_ref, coef_ref, prior_ref, out_ref):
    p = prior_ref[...]
    a = coef_ref[:, 0:1]
    d = coef_ref[:, 1:2]
    x1 = v0_ref[...] + a
    la1 = jnp.maximum(x1, p) + jnp.log1p(jnp.exp(-jnp.abs(x1 - p)))
    x2 = vt_ref[...] + d
    la2 = jnp.maximum(x2, p) + jnp.log1p(jnp.exp(-jnp.abs(x2 - p)))
    u = la1 + la2
    m = jnp.max(u, axis=1, keepdims=True)
    lse = m + jnp.log(jnp.sum(jnp.exp(u - m), axis=1, keepdims=True))
    out_ref[...] = u - lse


def _tc_dense(log_v0, log_vt, coef, prior, block=512):
    n, k = log_v0.shape
    grid = (n // block,)
    return pl.pallas_call(
        _tc_body,
        grid=grid,
        in_specs=[
            pl.BlockSpec((block, k), lambda i: (i, 0)),
            pl.BlockSpec((block, k), lambda i: (i, 0)),
            pl.BlockSpec((block, 2), lambda i: (i, 0)),
            pl.BlockSpec((1, k), lambda i: (0, 0)),
        ],
        out_specs=pl.BlockSpec((block, k), lambda i: (i, 0)),
        out_shape=jax.ShapeDtypeStruct((n, k), jnp.float32),
        compiler_params=pltpu.CompilerParams(
            dimension_semantics=("arbitrary",),
        ),
    )(log_v0, log_vt, coef, prior)


def kernel(log_v0, log_vt, t, batch, log_alphas_v, log_one_minus_alphas_v,
           log_alphas_cumprod_v, log_one_minus_alphas_cumprod_v, prior_probs):
    t_len = log_alphas_v.shape[0]
    pad_to = ((t_len + LANES - 1) // LANES) * LANES

    def padt(v):
        return jnp.concatenate(
            [v, jnp.zeros((pad_to - t_len,), v.dtype)]) if pad_to > t_len else v

    coef = _sc_coef(t, batch, padt(log_alphas_v),
                    padt(log_one_minus_alphas_v),
                    padt(log_alphas_cumprod_v),
                    padt(log_one_minus_alphas_cumprod_v))
    return _tc_dense(log_v0, log_vt, coef, prior_probs)


# speedup vs baseline: 5.6558x; 1.0045x over previous
import functools

import jax
import jax.numpy as jnp
from jax import lax
from jax.experimental import pallas as pl
from jax.experimental.pallas import tpu as pltpu
from jax.experimental.pallas import tpu_sc as plsc

NC = 2
NS = 16
LANES = 16
NW = NC * NS


_SC_CHUNK = 512


def _sc_coef_body(n_rows, t_len, n_b, t_hbm, batch_hbm, la_hbm, l1ma_hbm,
                  lac_hbm, l1mac_hbm, coef_hbm, t_v, batch_v, la_v, l1ma_v,
                  lac_v, l1mac_v, cba_v, cbd_v, coef_v):
    rows = n_rows // NW
    chunk = min(rows, _SC_CHUNK)
    wid = lax.axis_index("s") * NC + lax.axis_index("c")
    base = wid * rows
    pltpu.sync_copy(t_hbm, t_v)
    pltpu.sync_copy(la_hbm, la_v)
    pltpu.sync_copy(l1ma_hbm, l1ma_v)
    pltpu.sync_copy(lac_hbm, lac_v)
    pltpu.sync_copy(l1mac_hbm, l1mac_v)

    for j in range(n_b // LANES):
        tv = t_v[pl.ds(j * LANES, LANES)]
        tm1 = jnp.maximum(tv - 1, 0)
        a = plsc.load_gather(lac_v, [tm1])
        c = plsc.load_gather(l1mac_v, [tm1])
        d = plsc.load_gather(la_v, [tv])
        e = plsc.load_gather(l1ma_v, [tv])
        cba_v[pl.ds(j * LANES, LANES)] = a - c
        cbd_v[pl.ds(j * LANES, LANES)] = d - e

    zero = jnp.zeros((LANES,), jnp.int32)
    iota = lax.iota(jnp.int32, LANES)
    for ck in range(rows // chunk):
        cbase = base + ck * chunk
        pltpu.sync_copy(batch_hbm.at[pl.ds(cbase, chunk)], batch_v)

        @plsc.parallel_loop(0, chunk, LANES, unroll=4)
        def _fill(i):
            bv = batch_v[pl.ds(i, LANES)]
            av = plsc.load_gather(cba_v, [bv])
            dv = plsc.load_gather(cbd_v, [bv])
            ridx = i + iota
            plsc.store_scatter(coef_v, [ridx, zero], av)
            plsc.store_scatter(coef_v, [ridx, zero + 1], dv)

        pltpu.sync_copy(coef_v, coef_hbm.at[pl.ds(cbase, chunk)])


def _sc_coef(t, batch, la, l1ma, lac, l1mac):
    n_rows = batch.shape[0]
    b = t.shape[0]
    t_len = la.shape[0]
    rows = n_rows // NW
    mesh = plsc.VectorSubcoreMesh(core_axis_name="c", subcore_axis_name="s",
                                  num_cores=NC, num_subcores=NS)
    body = functools.partial(_sc_coef_body, n_rows, t_len, b)
    return pl.kernel(
        body,
        out_type=jax.ShapeDtypeStruct((n_rows, 2), jnp.float32),
        mesh=mesh,
        scratch_types=[
            pltpu.VMEM((b,), jnp.int32),
            pltpu.VMEM((min(rows, _SC_CHUNK),), jnp.int32),
            pltpu.VMEM((t_len,), jnp.float32),
            pltpu.VMEM((t_len,), jnp.float32),
            pltpu.VMEM((t_len,), jnp.float32),
            pltpu.VMEM((t_len,), jnp.float32),
            pltpu.VMEM((b,), jnp.float32),
            pltpu.VMEM((b,), jnp.float32),
            pltpu.VMEM((min(rows, _SC_CHUNK), 2), jnp.float32),
        ],
        compiler_params=pltpu.CompilerParams(needs_layout_passes=False),
    )(t, batch, la, l1ma, lac, l1mac)


def _tc_body(v0_ref, vt_ref, coef_ref, prior_ref, out_ref):
    p = prior_ref[...]
    a = coef_ref[:, 0:1]
    d = coef_ref[:, 1:2]
    x1 = v0_ref[...] + a
    la1 = jnp.maximum(x1, p) + jnp.log1p(jnp.exp(-jnp.abs(x1 - p)))
    x2 = vt_ref[...] + d
    la2 = jnp.maximum(x2, p) + jnp.log1p(jnp.exp(-jnp.abs(x2 - p)))
    u = la1 + la2
    m = jnp.max(u, axis=1, keepdims=True)
    lse = m + jnp.log(jnp.sum(jnp.exp(u - m), axis=1, keepdims=True))
    out_ref[...] = u - lse


def _tc_dense(log_v0, log_vt, coef, prior, block=512):
    n, k = log_v0.shape
    grid = (n // block,)
    return pl.pallas_call(
        _tc_body,
        grid=grid,
        in_specs=[
            pl.BlockSpec((block, k), lambda i: (i, 0)),
            pl.BlockSpec((block, k), lambda i: (i, 0)),
            pl.BlockSpec((block, 2), lambda i: (i, 0)),
            pl.BlockSpec((1, k), lambda i: (0, 0)),
        ],
        out_specs=pl.BlockSpec((block, k), lambda i: (i, 0)),
        out_shape=jax.ShapeDtypeStruct((n, k), jnp.float32),
        compiler_params=pltpu.CompilerParams(
            dimension_semantics=("arbitrary",),
        ),
    )(log_v0, log_vt, coef, prior)


def kernel(log_v0, log_vt, t, batch, log_alphas_v, log_one_minus_alphas_v,
           log_alphas_cumprod_v, log_one_minus_alphas_cumprod_v, prior_probs):
    t_len = log_alphas_v.shape[0]
    pad_to = ((t_len + LANES - 1) // LANES) * LANES

    def padt(v):
        return jnp.concatenate(
            [v, jnp.zeros((pad_to - t_len,), v.dtype)]) if pad_to > t_len else v

    coef = _sc_coef(t, batch, padt(log_alphas_v),
                    padt(log_one_minus_alphas_v),
                    padt(log_alphas_cumprod_v),
                    padt(log_one_minus_alphas_cumprod_v))
    return _tc_dense(log_v0, log_vt, coef, prior_probs)


# speedup vs baseline: 5.6718x; 1.0028x over previous
import functools

import jax
import jax.numpy as jnp
from jax import lax
from jax.experimental import pallas as pl
from jax.experimental.pallas import tpu as pltpu
from jax.experimental.pallas import tpu_sc as plsc

NC = 2
NS = 16
LANES = 16
NW = NC * NS


_SC_CHUNK = 512


def _sc_coef_body(t_len, n_b, t_hbm, la_hbm, l1ma_hbm,
                  lac_hbm, l1mac_hbm, coef_hbm, t_v, la_v, l1ma_v,
                  lac_v, l1mac_v, coef_v):
    pltpu.sync_copy(t_hbm, t_v)
    pltpu.sync_copy(la_hbm, la_v)
    pltpu.sync_copy(l1ma_hbm, l1ma_v)
    pltpu.sync_copy(lac_hbm, lac_v)
    pltpu.sync_copy(l1mac_hbm, l1mac_v)

    zero = jnp.zeros((LANES,), jnp.int32)
    iota = lax.iota(jnp.int32, LANES)
    for j in range(n_b // LANES):
        tv = t_v[pl.ds(j * LANES, LANES)]
        tm1 = jnp.maximum(tv - 1, 0)
        a = plsc.load_gather(lac_v, [tm1])
        c = plsc.load_gather(l1mac_v, [tm1])
        d = plsc.load_gather(la_v, [tv])
        e = plsc.load_gather(l1ma_v, [tv])
        ridx = j * LANES + iota
        plsc.store_scatter(coef_v, [ridx, zero], a - c)
        plsc.store_scatter(coef_v, [ridx, zero + 1], d - e)

    pltpu.sync_copy(coef_v, coef_hbm)


def _sc_coef(t, la, l1ma, lac, l1mac):
    b = t.shape[0]
    t_len = la.shape[0]
    mesh = plsc.VectorSubcoreMesh(core_axis_name="c", subcore_axis_name="s",
                                  num_cores=NC, num_subcores=NS)
    body = functools.partial(_sc_coef_body, t_len, b)
    return pl.kernel(
        body,
        out_type=jax.ShapeDtypeStruct((b, 128), jnp.float32),
        mesh=mesh,
        scratch_types=[
            pltpu.VMEM((b,), jnp.int32),
            pltpu.VMEM((t_len,), jnp.float32),
            pltpu.VMEM((t_len,), jnp.float32),
            pltpu.VMEM((t_len,), jnp.float32),
            pltpu.VMEM((t_len,), jnp.float32),
            pltpu.VMEM((b, 128), jnp.float32),
        ],
        compiler_params=pltpu.CompilerParams(needs_layout_passes=False),
    )(t, la, l1ma, lac, l1mac)


def _tc_body(v0_ref, vt_ref, batch_ref, cb_ref, prior_ref, out_ref):
    p = prior_ref[...]
    block, n_b = v0_ref.shape[0], cb_ref.shape[0]
    bt = batch_ref[0]
    oht = (bt == lax.broadcasted_iota(jnp.int32, (n_b, block), 0))
    coef = jax.lax.dot_general(oht.astype(jnp.float32), cb_ref[...],
                               (((0,), (0,)), ((), ())),
                               preferred_element_type=jnp.float32)
    a = coef[:, 0:1]
    d = coef[:, 1:2]
    x1 = v0_ref[...] + a
    la1 = jnp.maximum(x1, p) + jnp.log1p(jnp.exp(-jnp.abs(x1 - p)))
    x2 = vt_ref[...] + d
    la2 = jnp.maximum(x2, p) + jnp.log1p(jnp.exp(-jnp.abs(x2 - p)))
    u = la1 + la2
    m = jnp.max(u, axis=1, keepdims=True)
    lse = m + jnp.log(jnp.sum(jnp.exp(u - m), axis=1, keepdims=True))
    out_ref[...] = u - lse


def _tc_dense(log_v0, log_vt, batch, cb, prior, block=512):
    n, k = log_v0.shape
    n_b = cb.shape[0]
    grid = (n // block,)
    batch3 = batch.reshape(n // block, 1, block)
    return pl.pallas_call(
        _tc_body,
        grid=grid,
        in_specs=[
            pl.BlockSpec((block, k), lambda i: (i, 0)),
            pl.BlockSpec((block, k), lambda i: (i, 0)),
            pl.BlockSpec((1, 1, block), lambda i: (i, 0, 0)),
            pl.BlockSpec((n_b, 128), lambda i: (0, 0)),
            pl.BlockSpec((1, k), lambda i: (0, 0)),
        ],
        out_specs=pl.BlockSpec((block, k), lambda i: (i, 0)),
        out_shape=jax.ShapeDtypeStruct((n, k), jnp.float32),
        compiler_params=pltpu.CompilerParams(
            dimension_semantics=("arbitrary",),
        ),
    )(log_v0, log_vt, batch3, cb, prior)


def kernel(log_v0, log_vt, t, batch, log_alphas_v, log_one_minus_alphas_v,
           log_alphas_cumprod_v, log_one_minus_alphas_cumprod_v, prior_probs):
    t_len = log_alphas_v.shape[0]
    pad_to = ((t_len + LANES - 1) // LANES) * LANES

    def padt(v):
        return jnp.concatenate(
            [v, jnp.zeros((pad_to - t_len,), v.dtype)]) if pad_to > t_len else v

    cb = _sc_coef(t, padt(log_alphas_v),
                  padt(log_one_minus_alphas_v),
                  padt(log_alphas_cumprod_v),
                  padt(log_one_minus_alphas_cumprod_v))
    return _tc_dense(log_v0, log_vt, batch, cb, prior_probs)


# speedup vs baseline: 5.8554x; 1.0324x over previous
import functools

import jax
import jax.numpy as jnp
from jax import lax
from jax.experimental import pallas as pl
from jax.experimental.pallas import tpu as pltpu
from jax.experimental.pallas import tpu_sc as plsc

NC = 2
NS = 16
LANES = 16
NW = NC * NS


_SC_CHUNK = 512


def _sc_coef_body(t_len, n_b, t_hbm, la_hbm, l1ma_hbm,
                  lac_hbm, l1mac_hbm, coef_hbm, t_v, la_v, l1ma_v,
                  lac_v, l1mac_v, coef_v):
    pltpu.sync_copy(t_hbm, t_v)
    pltpu.sync_copy(la_hbm, la_v.at[pl.ds(0, t_len)])
    pltpu.sync_copy(l1ma_hbm, l1ma_v.at[pl.ds(0, t_len)])
    pltpu.sync_copy(lac_hbm, lac_v.at[pl.ds(0, t_len)])
    pltpu.sync_copy(l1mac_hbm, l1mac_v.at[pl.ds(0, t_len)])

    zero = jnp.zeros((LANES,), jnp.int32)
    iota = lax.iota(jnp.int32, LANES)
    for j in range(n_b // LANES):
        tv = t_v[pl.ds(j * LANES, LANES)]
        tm1 = jnp.maximum(tv - 1, 0)
        a = plsc.load_gather(lac_v, [tm1])
        c = plsc.load_gather(l1mac_v, [tm1])
        d = plsc.load_gather(la_v, [tv])
        e = plsc.load_gather(l1ma_v, [tv])
        ridx = j * LANES + iota
        plsc.store_scatter(coef_v, [ridx, zero], a - c)
        plsc.store_scatter(coef_v, [ridx, zero + 1], d - e)

    pltpu.sync_copy(coef_v, coef_hbm)


def _sc_coef(t, la, l1ma, lac, l1mac):
    b = t.shape[0]
    t_len = la.shape[0]
    t_pad = ((t_len + LANES - 1) // LANES) * LANES
    mesh = plsc.VectorSubcoreMesh(core_axis_name="c", subcore_axis_name="s",
                                  num_cores=NC, num_subcores=NS)
    body = functools.partial(_sc_coef_body, t_len, b)
    return pl.kernel(
        body,
        out_type=jax.ShapeDtypeStruct((b, 128), jnp.float32),
        mesh=mesh,
        scratch_types=[
            pltpu.VMEM((b,), jnp.int32),
            pltpu.VMEM((t_pad,), jnp.float32),
            pltpu.VMEM((t_pad,), jnp.float32),
            pltpu.VMEM((t_pad,), jnp.float32),
            pltpu.VMEM((t_pad,), jnp.float32),
            pltpu.VMEM((b, 128), jnp.float32),
        ],
        compiler_params=pltpu.CompilerParams(needs_layout_passes=False),
    )(t, la, l1ma, lac, l1mac)


def _tc_body(v0_ref, vt_ref, batch_ref, cb_ref, prior_ref, out_ref):
    p = prior_ref[...]
    block, n_b = v0_ref.shape[0], cb_ref.shape[0]
    bt = batch_ref[0]
    oht = (bt == lax.broadcasted_iota(jnp.int32, (n_b, block), 0))
    coef = jax.lax.dot_general(oht.astype(jnp.float32), cb_ref[...],
                               (((0,), (0,)), ((), ())),
                               preferred_element_type=jnp.float32)
    a = coef[:, 0:1]
    d = coef[:, 1:2]
    x1 = v0_ref[...] + a
    la1 = jnp.maximum(x1, p) + jnp.log1p(jnp.exp(-jnp.abs(x1 - p)))
    x2 = vt_ref[...] + d
    la2 = jnp.maximum(x2, p) + jnp.log1p(jnp.exp(-jnp.abs(x2 - p)))
    u = la1 + la2
    m = jnp.max(u, axis=1, keepdims=True)
    lse = m + jnp.log(jnp.sum(jnp.exp(u - m), axis=1, keepdims=True))
    out_ref[...] = u - lse


def _tc_dense(log_v0, log_vt, batch, cb, prior, block=1024):
    n, k = log_v0.shape
    n_b = cb.shape[0]
    grid = (n // block,)
    batch3 = batch.reshape(n // block, 1, block)
    return pl.pallas_call(
        _tc_body,
        grid=grid,
        in_specs=[
            pl.BlockSpec((block, k), lambda i: (i, 0)),
            pl.BlockSpec((block, k), lambda i: (i, 0)),
            pl.BlockSpec((1, 1, block), lambda i: (i, 0, 0)),
            pl.BlockSpec((n_b, 128), lambda i: (0, 0)),
            pl.BlockSpec((1, k), lambda i: (0, 0)),
        ],
        out_specs=pl.BlockSpec((block, k), lambda i: (i, 0)),
        out_shape=jax.ShapeDtypeStruct((n, k), jnp.float32),
        compiler_params=pltpu.CompilerParams(
            dimension_semantics=("arbitrary",),
        ),
    )(log_v0, log_vt, batch3, cb, prior)


def kernel(log_v0, log_vt, t, batch, log_alphas_v, log_one_minus_alphas_v,
           log_alphas_cumprod_v, log_one_minus_alphas_cumprod_v, prior_probs):
    cb = _sc_coef(t, log_alphas_v, log_one_minus_alphas_v,
                  log_alphas_cumprod_v, log_one_minus_alphas_cumprod_v)
    return _tc_dense(log_v0, log_vt, batch, cb, prior_probs)


# speedup vs baseline: 5.8825x; 1.0046x over previous
import functools

import jax
import jax.numpy as jnp
from jax import lax
from jax.experimental import pallas as pl
from jax.experimental.pallas import tpu as pltpu
from jax.experimental.pallas import tpu_sc as plsc

NC = 2
NS = 16
LANES = 16
NW = NC * NS


_SC_CHUNK = 512


def _sc_coef_body(t_len, n_b, t_hbm, la_hbm, l1ma_hbm,
                  lac_hbm, l1mac_hbm, coef_hbm, t_v, la_v, l1ma_v,
                  lac_v, l1mac_v, coef_v):
    pltpu.sync_copy(t_hbm, t_v)
    pltpu.sync_copy(la_hbm, la_v.at[pl.ds(0, t_len)])
    pltpu.sync_copy(l1ma_hbm, l1ma_v.at[pl.ds(0, t_len)])
    pltpu.sync_copy(lac_hbm, lac_v.at[pl.ds(0, t_len)])
    pltpu.sync_copy(l1mac_hbm, l1mac_v.at[pl.ds(0, t_len)])

    zero = jnp.zeros((LANES,), jnp.int32)
    iota = lax.iota(jnp.int32, LANES)
    for j in range(n_b // LANES):
        tv = t_v[pl.ds(j * LANES, LANES)]
        tm1 = jnp.maximum(tv - 1, 0)
        a = plsc.load_gather(lac_v, [tm1])
        c = plsc.load_gather(l1mac_v, [tm1])
        d = plsc.load_gather(la_v, [tv])
        e = plsc.load_gather(l1ma_v, [tv])
        ridx = j * LANES + iota
        plsc.store_scatter(coef_v, [ridx, zero], a - c)
        plsc.store_scatter(coef_v, [ridx, zero + 1], d - e)

    pltpu.sync_copy(coef_v, coef_hbm)


def _sc_coef(t, la, l1ma, lac, l1mac):
    b = t.shape[0]
    t_len = la.shape[0]
    t_pad = ((t_len + LANES - 1) // LANES) * LANES
    mesh = plsc.VectorSubcoreMesh(core_axis_name="c", subcore_axis_name="s",
                                  num_cores=NC, num_subcores=NS)
    body = functools.partial(_sc_coef_body, t_len, b)
    return pl.kernel(
        body,
        out_type=jax.ShapeDtypeStruct((b, 128), jnp.float32),
        mesh=mesh,
        scratch_types=[
            pltpu.VMEM((b,), jnp.int32),
            pltpu.VMEM((t_pad,), jnp.float32),
            pltpu.VMEM((t_pad,), jnp.float32),
            pltpu.VMEM((t_pad,), jnp.float32),
            pltpu.VMEM((t_pad,), jnp.float32),
            pltpu.VMEM((b, 128), jnp.float32),
        ],
        compiler_params=pltpu.CompilerParams(needs_layout_passes=False),
    )(t, la, l1ma, lac, l1mac)


def _tc_body(v0_ref, vt_ref, batch_ref, cb_ref, prior_ref, out_ref):
    p = prior_ref[...]
    block, n_b = v0_ref.shape[0], cb_ref.shape[0]
    bt = batch_ref[0]
    oht = (bt == lax.broadcasted_iota(jnp.int32, (n_b, block), 0))
    coef = jax.lax.dot_general(oht.astype(jnp.float32), cb_ref[...],
                               (((0,), (0,)), ((), ())),
                               preferred_element_type=jnp.float32)
    a = coef[:, 0:1]
    d = coef[:, 1:2]
    x1 = v0_ref[...] + a
    la1 = jnp.maximum(x1, p) + jnp.log1p(jnp.exp(-jnp.abs(x1 - p)))
    x2 = vt_ref[...] + d
    la2 = jnp.maximum(x2, p) + jnp.log1p(jnp.exp(-jnp.abs(x2 - p)))
    u = la1 + la2
    m = jnp.max(u, axis=1, keepdims=True)
    lse = m + jnp.log(jnp.sum(jnp.exp(u - m), axis=1, keepdims=True))
    out_ref[...] = u - lse


def _tc_dense(log_v0, log_vt, batch, cb, prior, block=2048):
    n, k = log_v0.shape
    n_b = cb.shape[0]
    grid = (n // block,)
    batch3 = batch.reshape(n // block, 1, block)
    return pl.pallas_call(
        _tc_body,
        grid=grid,
        in_specs=[
            pl.BlockSpec((block, k), lambda i: (i, 0)),
            pl.BlockSpec((block, k), lambda i: (i, 0)),
            pl.BlockSpec((1, 1, block), lambda i: (i, 0, 0)),
            pl.BlockSpec((n_b, 128), lambda i: (0, 0)),
            pl.BlockSpec((1, k), lambda i: (0, 0)),
        ],
        out_specs=pl.BlockSpec((block, k), lambda i: (i, 0)),
        out_shape=jax.ShapeDtypeStruct((n, k), jnp.float32),
        compiler_params=pltpu.CompilerParams(
            dimension_semantics=("arbitrary",),
        ),
    )(log_v0, log_vt, batch3, cb, prior)


def kernel(log_v0, log_vt, t, batch, log_alphas_v, log_one_minus_alphas_v,
           log_alphas_cumprod_v, log_one_minus_alphas_cumprod_v, prior_probs):
    cb = _sc_coef(t, log_alphas_v, log_one_minus_alphas_v,
                  log_alphas_cumprod_v, log_one_minus_alphas_cumprod_v)
    return _tc_dense(log_v0, log_vt, batch, cb, prior_probs)


# speedup vs baseline: 5.9694x; 1.0148x over previous
import functools

import jax
import jax.numpy as jnp
from jax import lax
from jax.experimental import pallas as pl
from jax.experimental.pallas import tpu as pltpu
from jax.experimental.pallas import tpu_sc as plsc

NC = 2
NS = 16
LANES = 16
NW = NC * NS


_SC_CHUNK = 512


def _sc_coef_body(t_len, n_b, t_hbm, la_hbm, l1ma_hbm,
                  lac_hbm, l1mac_hbm, coef_hbm, t_v, la_v, l1ma_v,
                  lac_v, l1mac_v, coef_v):
    wid = lax.axis_index("s") * NC + lax.axis_index("c")

    @pl.when(wid < n_b // LANES)
    def _():
        pltpu.sync_copy(t_hbm, t_v)
        pltpu.sync_copy(la_hbm, la_v.at[pl.ds(0, t_len)])
        pltpu.sync_copy(l1ma_hbm, l1ma_v.at[pl.ds(0, t_len)])
        pltpu.sync_copy(lac_hbm, lac_v.at[pl.ds(0, t_len)])
        pltpu.sync_copy(l1mac_hbm, l1mac_v.at[pl.ds(0, t_len)])

        zero = jnp.zeros((LANES,), jnp.int32)
        iota = lax.iota(jnp.int32, LANES)
        tv = t_v[pl.ds(wid * LANES, LANES)]
        tm1 = jnp.maximum(tv - 1, 0)
        a = plsc.load_gather(lac_v, [tm1])
        c = plsc.load_gather(l1mac_v, [tm1])
        d = plsc.load_gather(la_v, [tv])
        e = plsc.load_gather(l1ma_v, [tv])
        plsc.store_scatter(coef_v, [iota, zero], a - c)
        plsc.store_scatter(coef_v, [iota, zero + 1], d - e)
        pltpu.sync_copy(coef_v, coef_hbm.at[pl.ds(wid * LANES, LANES)])


def _sc_coef(t, la, l1ma, lac, l1mac):
    b = t.shape[0]
    t_len = la.shape[0]
    t_pad = ((t_len + LANES - 1) // LANES) * LANES
    mesh = plsc.VectorSubcoreMesh(core_axis_name="c", subcore_axis_name="s",
                                  num_cores=NC, num_subcores=NS)
    body = functools.partial(_sc_coef_body, t_len, b)
    return pl.kernel(
        body,
        out_type=jax.ShapeDtypeStruct((b, 128), jnp.float32),
        mesh=mesh,
        scratch_types=[
            pltpu.VMEM((b,), jnp.int32),
            pltpu.VMEM((t_pad,), jnp.float32),
            pltpu.VMEM((t_pad,), jnp.float32),
            pltpu.VMEM((t_pad,), jnp.float32),
            pltpu.VMEM((t_pad,), jnp.float32),
            pltpu.VMEM((LANES, 128), jnp.float32),
        ],
        compiler_params=pltpu.CompilerParams(needs_layout_passes=False),
    )(t, la, l1ma, lac, l1mac)


def _tc_body(v0_ref, vt_ref, batch_ref, cb_ref, prior_ref, out_ref):
    p = prior_ref[...]
    block, n_b = v0_ref.shape[0], cb_ref.shape[0]
    bt = batch_ref[0]
    oht = (bt == lax.broadcasted_iota(jnp.int32, (n_b, block), 0))
    coef = jax.lax.dot_general(oht.astype(jnp.float32), cb_ref[...],
                               (((0,), (0,)), ((), ())),
                               preferred_element_type=jnp.float32)
    a = coef[:, 0:1]
    d = coef[:, 1:2]
    x1 = v0_ref[...] + a
    la1 = jnp.maximum(x1, p) + jnp.log1p(jnp.exp(-jnp.abs(x1 - p)))
    x2 = vt_ref[...] + d
    la2 = jnp.maximum(x2, p) + jnp.log1p(jnp.exp(-jnp.abs(x2 - p)))
    u = la1 + la2
    m = jnp.max(u, axis=1, keepdims=True)
    lse = m + jnp.log(jnp.sum(jnp.exp(u - m), axis=1, keepdims=True))
    out_ref[...] = u - lse


def _tc_dense(log_v0, log_vt, batch, cb, prior, block=2048):
    n, k = log_v0.shape
    n_b = cb.shape[0]
    grid = (n // block,)
    batch3 = batch.reshape(n // block, 1, block)
    return pl.pallas_call(
        _tc_body,
        grid=grid,
        in_specs=[
            pl.BlockSpec((block, k), lambda i: (i, 0)),
            pl.BlockSpec((block, k), lambda i: (i, 0)),
            pl.BlockSpec((1, 1, block), lambda i: (i, 0, 0)),
            pl.BlockSpec((n_b, 128), lambda i: (0, 0)),
            pl.BlockSpec((1, k), lambda i: (0, 0)),
        ],
        out_specs=pl.BlockSpec((block, k), lambda i: (i, 0)),
        out_shape=jax.ShapeDtypeStruct((n, k), jnp.float32),
        compiler_params=pltpu.CompilerParams(
            dimension_semantics=("arbitrary",),
        ),
    )(log_v0, log_vt, batch3, cb, prior)


def kernel(log_v0, log_vt, t, batch, log_alphas_v, log_one_minus_alphas_v,
           log_alphas_cumprod_v, log_one_minus_alphas_cumprod_v, prior_probs):
    cb = _sc_coef(t, log_alphas_v, log_one_minus_alphas_v,
                  log_alphas_cumprod_v, log_one_minus_alphas_cumprod_v)
    return _tc_dense(log_v0, log_vt, batch, cb, prior_probs)
